# baseline (speedup 1.0000x reference)
import functools

import jax
import jax.numpy as jnp
from jax import lax
from jax.experimental import pallas as pl
from jax.experimental.pallas import tpu as pltpu
from jax.experimental.pallas import tpu_sc as plsc

_B, _S = 4096, 200
_D = 64
_V = 1000000
_NC, _NS = 2, 16
_NW = _NC * _NS
_CHUNK = 128
_NBJ = _B // _CHUNK
_NCHUNK = _S * _NBJ // _NW
_NBUF = 4

_NFULL = _V // _CHUNK
_TAIL = _V - _NFULL * _CHUNK
_JBASE = _NFULL // _NW
_JREM = _NFULL % _NW

_mesh = plsc.VectorSubcoreMesh(core_axis_name="c", subcore_axis_name="s")


@functools.partial(
    pl.kernel,
    out_type=jax.ShapeDtypeStruct((_V // 2, 2 * _D), jnp.float32),
    mesh=_mesh,
    scratch_types=[
        pltpu.VMEM((2, _D, _CHUNK), jnp.float32),
        pltpu.VMEM((2, _CHUNK // 2, 2 * _D), jnp.float32),
        pltpu.SemaphoreType.DMA,
        pltpu.SemaphoreType.DMA,
    ],
    compiler_params=pltpu.CompilerParams(needs_layout_passes=False),
)
def _transpose_kernel(wt_hbm, tail_hbm, tab_hbm, in_v, out_v, sem_i, sem_o):
    wid = lax.axis_index("s") * _NC + lax.axis_index("c")
    j_lo = wid * _JBASE + jnp.minimum(wid, _JREM)
    n_j = _JBASE + jnp.where(wid < _JREM, 1, 0)

    rows_half = [lax.shift_right_logical(lax.iota(jnp.int32, 16) + 16 * g, 1)
                 for g in range(8)]
    colpar = lax.shift_left(
        lax.bitwise_and(lax.iota(jnp.int32, 16), 1), 6
    )

    def in_start(j, slot):
        pltpu.async_copy(
            wt_hbm.at[:, pl.ds(j * _CHUNK, _CHUNK)], in_v.at[slot], sem_i
        )

    def in_wait(slot):
        pltpu.make_async_copy(
            wt_hbm.at[:, pl.ds(0, _CHUNK)], in_v.at[slot], sem_i
        ).wait()

    def transpose(slot):
        iref = in_v.at[slot]
        oref = out_v.at[slot]

        def body(d, _):
            col = colpar + d
            for g in range(8):
                v = iref[d, pl.ds(16 * g, 16)]
                plsc.store_scatter(oref, [rows_half[g], col], v)
            return ()

        lax.fori_loop(0, _D, body, ())

    def out_start(j, slot):
        pltpu.async_copy(
            out_v.at[slot], tab_hbm.at[pl.ds(j * (_CHUNK // 2), _CHUNK // 2)],
            sem_o,
        )

    def out_wait(slot):
        pltpu.make_async_copy(
            tab_hbm.at[pl.ds(0, _CHUNK // 2)], out_v.at[slot], sem_o
        ).wait()

    in_start(j_lo, 0)

    def body(t, _):
        for p in range(2):
            i = 2 * t + p
            j = j_lo + i

            @pl.when(i < n_j)
            def _():
                @pl.when(i + 1 < n_j)
                def _():
                    in_start(j + 1, 1 - p)

                in_wait(p)
                transpose(p)

                @pl.when(i >= 2)
                def _():
                    out_wait(p)

                out_start(j, p)

        return ()

    lax.fori_loop(0, (_JBASE + 2) // 2, body, ())

    @pl.when(n_j >= 2)
    def _():
        out_wait(n_j % 2)

    out_wait(1 - n_j % 2)

    @pl.when(wid == _NW - 1)
    def _():
        pltpu.sync_copy(tail_hbm, in_v.at[0, pl.ds(0, _TAIL // 2)])
        pltpu.sync_copy(
            in_v.at[0, pl.ds(0, _TAIL // 2)],
            tab_hbm.at[pl.ds(_NFULL * (_CHUNK // 2), _TAIL // 2)],
        )


@functools.partial(
    pl.kernel,
    out_type=jax.ShapeDtypeStruct((_S, _D, _B), jnp.float32),
    mesh=_mesh,
    scratch_types=[
        pltpu.VMEM((_NBUF, _CHUNK), jnp.int32),
        pltpu.VMEM((_NBUF, _CHUNK), jnp.int32),
        pltpu.VMEM((_NBUF, _CHUNK), jnp.int32),
        pltpu.VMEM((_NBUF, _CHUNK, 2 * _D), jnp.float32),
        pltpu.VMEM((_NBUF, _D, _CHUNK), jnp.float32),
        pltpu.SemaphoreType.DMA,
        pltpu.SemaphoreType.DMA,
        pltpu.SemaphoreType.DMA,
    ],
    compiler_params=pltpu.CompilerParams(needs_layout_passes=False),
)
def _gather_kernel(tok_hbm, tab_hbm, out_hbm, tok_v, pair_v, parb_v, gath_v,
                   blk_v, sem_t, sem_g, sem_s):
    wid = lax.axis_index("s") * _NC + lax.axis_index("c")
    c0 = wid * _NCHUNK

    rows = [lax.iota(jnp.int32, 16) + 16 * g for g in range(8)]

    def chunk_sbj(i):
        c = c0 + i
        return c // _NBJ, c % _NBJ

    def tok_start(i, slot):
        s, bj = chunk_sbj(i)
        pltpu.async_copy(
            tok_hbm.at[s, pl.ds(bj * _CHUNK, _CHUNK)], tok_v.at[slot], sem_t
        )

    def tok_wait(slot):
        pltpu.make_async_copy(
            tok_hbm.at[0, pl.ds(0, _CHUNK)], tok_v.at[slot], sem_t
        ).wait()

    def pair_compute(slot):
        for g in range(8):
            t = tok_v[slot, pl.ds(16 * g, 16)]
            pair_v[slot, pl.ds(16 * g, 16)] = lax.shift_right_logical(t, 1)
            parb_v[slot, pl.ds(16 * g, 16)] = lax.shift_left(
                lax.bitwise_and(t, 1), 6
            )

    def gather_start(slot):
        pltpu.async_copy(tab_hbm.at[pair_v.at[slot]], gath_v.at[slot], sem_g)

    def gather_wait(slot):
        pltpu.make_async_copy(
            tab_hbm.at[pl.ds(0, _CHUNK)], gath_v.at[slot], sem_g
        ).wait()

    def assemble(slot):
        gref = gath_v.at[slot]
        bref = blk_v.at[slot]
        parbs = tuple(parb_v[slot, pl.ds(16 * g, 16)] for g in range(8))

        def body(d, carry):
            for g in range(8):
                col = carry[g] + d
                v = plsc.load_gather(gref, [rows[g], col])
                bref[d, pl.ds(16 * g, 16)] = v
            return carry

        lax.fori_loop(0, _D, body, parbs)

    def store_start(i, slot):
        s, bj = chunk_sbj(i)
        pltpu.async_copy(
            blk_v.at[slot], out_hbm.at[s, :, pl.ds(bj * _CHUNK, _CHUNK)], sem_s
        )

    def store_wait(slot):
        pltpu.make_async_copy(
            out_hbm.at[0, :, pl.ds(0, _CHUNK)], blk_v.at[slot], sem_s
        ).wait()

    for j in range(_NBUF):
        tok_start(j, j)
    for j in range(3):
        tok_wait(j)
        pair_compute(j)
        gather_start(j)

    def body(t, _):
        for p in range(_NBUF):
            i = t * _NBUF + p

            gather_wait(p)

            @pl.when(i + _NBUF < _NCHUNK)
            def _():
                tok_start(i + _NBUF, p)

            @pl.when(i >= 2)
            def _():
                store_wait((p + 2) % _NBUF)

            assemble(p)
            store_start(i, p)

            @pl.when(i + 3 < _NCHUNK)
            def _():
                tok_wait((p + 3) % _NBUF)
                pair_compute((p + 3) % _NBUF)
                gather_start((p + 3) % _NBUF)

        return ()

    lax.fori_loop(0, _NCHUNK // _NBUF, body, ())
    store_wait((_NCHUNK - 2) % _NBUF)
    store_wait((_NCHUNK - 1) % _NBUF)


def kernel(token_ids, weights):
    tok_t = token_ids.T
    w_t = weights.T
    tail = weights[_NFULL * _CHUNK:].reshape(_TAIL // 2, 2 * _D)
    tab = _transpose_kernel(w_t, tail)
    out_t = _gather_kernel(tok_t, tab)
    return jnp.transpose(out_t, (2, 0, 1))
